# baseline (speedup 1.0000x reference)
import functools

import jax
import jax.numpy as jnp
from jax import lax
from jax.experimental import pallas as pl
from jax.experimental.pallas import tpu as pltpu
from jax.experimental.pallas import tpu_sc as plsc

_NC = 2
_NS = 16
_NW = _NC * _NS
_NB = 2


def _transpose_kernel(d_model, vocab, d_pad):
    blk = 128
    n_full = vocab // blk
    mesh = plsc.VectorSubcoreMesh(core_axis_name="c", subcore_axis_name="s")

    @functools.partial(
        pl.kernel,
        mesh=mesh,
        out_type=jax.ShapeDtypeStruct((vocab, d_pad), jnp.float32),
        scratch_types=[
            pltpu.VMEM((_NB, d_model, blk), jnp.float32),
            pltpu.VMEM((_NB, blk, d_pad), jnp.float32),
            pltpu.SemaphoreType.DMA,
            pltpu.SemaphoreType.DMA,
        ],
        compiler_params=pltpu.CompilerParams(use_tc_tiling_on_sc=True,
                                             needs_layout_passes=False),
    )
    def k(wt_hbm, wt_tail_hbm, out_hbm, in_v, tr_v, isem, osem):
        wid = lax.axis_index("s") * _NC + lax.axis_index("c")
        n_mine = (n_full // _NW) + jnp.where(wid < (n_full % _NW), 1, 0)

        def col0_of(t):
            return (wid + t * _NW) * blk

        def load_args(t, b):
            return (wt_hbm.at[:, pl.ds(col0_of(t), blk)], in_v.at[b], isem)

        def store_args(t, b):
            return (tr_v.at[b], out_hbm.at[pl.ds(col0_of(t), blk)], osem)

        lane = lax.iota(jnp.int32, 16)

        def transpose_block(b, ncols):
            @pl.loop(0, ncols, unroll=4)
            def _(j):
                row_j = jnp.full((16,), j, jnp.int32)
                for d0 in range(0, d_model, 16):
                    v = plsc.load_gather(in_v.at[b], [lane + d0, row_j])
                    plsc.store_scatter(tr_v.at[b], [row_j, lane + d0], v)

        pltpu.async_copy(*load_args(0, 0))
        n_iters = n_full // _NW + 1

        @pl.loop(0, n_iters, step=_NB)
        def _(t0):
            for b in range(_NB):
                t = t0 + b

                @pl.when(t < n_mine)
                def _():
                    pltpu.make_async_copy(*load_args(t, b)).wait()

                    @pl.when(t + 1 < n_mine)
                    def _():
                        pltpu.async_copy(*load_args(t + 1, (b + 1) % _NB))

                    @pl.when(t >= _NB)
                    def _():
                        pltpu.make_async_copy(*store_args(t - _NB, b)).wait()

                    transpose_block(b, blk)
                    pltpu.async_copy(*store_args(t, b))

        for _ in range(_NB):
            pltpu.make_async_copy(tr_v.at[0],
                                  out_hbm.at[pl.ds(0, blk)], osem).wait()

        if vocab % blk:
            @pl.when(wid == 0)
            def _():
                pltpu.sync_copy(wt_tail_hbm, in_v.at[0])
                transpose_block(0, blk)
                pltpu.sync_copy(tr_v.at[0],
                                out_hbm.at[pl.ds(vocab - blk, blk)])

    return k


def _gather_kernel(n_total, d_model, d_pad, chunk):
    b_per_w = n_total // _NW
    n_chunks = b_per_w // chunk
    mesh = plsc.VectorSubcoreMesh(core_axis_name="c", subcore_axis_name="s")

    @functools.partial(
        pl.kernel,
        mesh=mesh,
        out_type=jax.ShapeDtypeStruct((n_total, d_pad), jnp.float32),
        scratch_types=[
            pltpu.VMEM((b_per_w,), jnp.int32),
            pltpu.VMEM((_NB, chunk, d_model), jnp.float32),
            pltpu.SemaphoreType.DMA,
            pltpu.SemaphoreType.DMA,
        ],
        compiler_params=pltpu.CompilerParams(use_tc_tiling_on_sc=False),
    )
    def k(idx_hbm, table_hbm, out_hbm, idx_v, rows_v, gsem, ssem):
        wid = lax.axis_index("s") * _NC + lax.axis_index("c")
        base = wid * b_per_w

        def gather_args(i, b):
            return (table_hbm.at[idx_v.at[pl.ds(i * chunk, chunk)]],
                    rows_v.at[b], gsem)

        def store_args(i, b):
            return (rows_v.at[b],
                    out_hbm.at[pl.ds(base + i * chunk, chunk),
                               pl.ds(0, d_model)], ssem)

        def gather(i, b):
            pltpu.async_copy(*gather_args(i, b))

        def gather_wait(i, b):
            pltpu.make_async_copy(*gather_args(i, b)).wait()

        def store(i, b):
            pltpu.async_copy(*store_args(i, b))

        def store_wait(i, b):
            pltpu.make_async_copy(*store_args(i, b)).wait()

        pltpu.sync_copy(idx_hbm.at[pl.ds(base, b_per_w)], idx_v)
        for b in range(_NB):
            gather(b, b)

        @pl.loop(0, n_chunks - _NB, step=_NB)
        def _(i0):
            for b in range(_NB):
                i = i0 + b
                gather_wait(i, b)
                store(i, b)
                store_wait(i, b)
                gather(i + _NB, b)

        for b in range(_NB):
            i = n_chunks - _NB + b
            gather_wait(i, b)
            store(i, b)
        for b in range(_NB):
            store_wait(n_chunks - _NB + b, b)

    return k


def kernel(ids, emb_weight):
    batch, hist = ids.shape
    vocab, d_model = emb_weight.shape
    d_pad = 128
    n_total = batch * hist
    rows_per_pad = d_pad // d_model
    idx = ids.reshape(n_total).astype(jnp.int32) * rows_per_pad
    wt = emb_weight.T
    table_p = _transpose_kernel(d_model, vocab, d_pad)(
        wt, lax.slice(wt, (0, vocab - 128), (d_model, vocab)))
    table_v = table_p.reshape(vocab * rows_per_pad, d_model)
    out = _gather_kernel(n_total, d_model, d_pad, 800)(idx, table_v)
    return out[:, :d_model].reshape(batch, hist, d_model)


# speedup vs baseline: 2.6960x; 2.6960x over previous
import functools

import jax
import jax.numpy as jnp
from jax import lax
from jax.experimental import pallas as pl
from jax.experimental.pallas import tpu as pltpu
from jax.experimental.pallas import tpu_sc as plsc

_NC = 2
_NS = 16
_NW = _NC * _NS
_NB = 2


def _pack_kernel(d_model, vocab, d_pad, cols):
    grid = (vocab + cols - 1) // cols

    def body(wt_ref, out_ref):
        t = jnp.transpose(wt_ref[...], (1, 0))
        out_ref[...] = jnp.concatenate(
            [t, jnp.zeros((cols, d_pad - d_model), jnp.float32)], axis=1)

    return pl.pallas_call(
        body,
        grid=(grid,),
        in_specs=[pl.BlockSpec((d_model, cols), lambda i: (0, i))],
        out_specs=pl.BlockSpec((cols, d_pad), lambda i: (i, 0)),
        out_shape=jax.ShapeDtypeStruct((vocab, d_pad), jnp.float32),
    )


def _gather_kernel(n_total, d_model, d_pad, chunk):
    b_per_w = n_total // _NW
    n_chunks = b_per_w // chunk
    mesh = plsc.VectorSubcoreMesh(core_axis_name="c", subcore_axis_name="s")

    @functools.partial(
        pl.kernel,
        mesh=mesh,
        out_type=jax.ShapeDtypeStruct((n_total, d_pad), jnp.float32),
        scratch_types=[
            pltpu.VMEM((b_per_w,), jnp.int32),
            pltpu.VMEM((_NB, chunk, d_model), jnp.float32),
            pltpu.SemaphoreType.DMA,
            pltpu.SemaphoreType.DMA,
        ],
        compiler_params=pltpu.CompilerParams(use_tc_tiling_on_sc=False),
    )
    def k(idx_hbm, table_hbm, out_hbm, idx_v, rows_v, gsem, ssem):
        wid = lax.axis_index("s") * _NC + lax.axis_index("c")
        base = wid * b_per_w

        def gather_args(i, b):
            return (table_hbm.at[idx_v.at[pl.ds(i * chunk, chunk)]],
                    rows_v.at[b], gsem)

        def store_args(i, b):
            return (rows_v.at[b],
                    out_hbm.at[pl.ds(base + i * chunk, chunk),
                               pl.ds(0, d_model)], ssem)

        def gather(i, b):
            pltpu.async_copy(*gather_args(i, b))

        def gather_wait(i, b):
            pltpu.make_async_copy(*gather_args(i, b)).wait()

        def store(i, b):
            pltpu.async_copy(*store_args(i, b))

        def store_wait(i, b):
            pltpu.make_async_copy(*store_args(i, b)).wait()

        pltpu.sync_copy(idx_hbm.at[pl.ds(base, b_per_w)], idx_v)
        for b in range(_NB):
            gather(b, b)

        @pl.loop(0, n_chunks - _NB, step=_NB)
        def _(i0):
            for b in range(_NB):
                i = i0 + b
                gather_wait(i, b)
                store(i, b)
                store_wait(i, b)
                gather(i + _NB, b)

        for b in range(_NB):
            i = n_chunks - _NB + b
            gather_wait(i, b)
            store(i, b)
        for b in range(_NB):
            store_wait(n_chunks - _NB + b, b)

    return k


def kernel(ids, emb_weight):
    batch, hist = ids.shape
    vocab, d_model = emb_weight.shape
    d_pad = 128
    n_total = batch * hist
    rows_per_pad = d_pad // d_model
    idx = ids.reshape(n_total).astype(jnp.int32) * rows_per_pad
    table_p = _pack_kernel(d_model, vocab, d_pad, 4096)(emb_weight.T)
    table_v = table_p.reshape(vocab * rows_per_pad, d_model)
    out = _gather_kernel(n_total, d_model, d_pad, 800)(idx, table_v)
    return out[:, :d_model].reshape(batch, hist, d_model)


# speedup vs baseline: 3.0110x; 1.1168x over previous
import functools

import jax
import jax.numpy as jnp
from jax import lax
from jax.experimental import pallas as pl
from jax.experimental.pallas import tpu as pltpu
from jax.experimental.pallas import tpu_sc as plsc

_NC = 2
_NS = 16
_NW = _NC * _NS
_NB = 2


def _pack_kernel(d_model, vocab, d_pad, cols):
    grid = (vocab + cols - 1) // cols

    def body(wt_ref, out_ref):
        t = jnp.transpose(wt_ref[...], (1, 0))
        out_ref[...] = jnp.concatenate(
            [t, jnp.zeros((cols, d_pad - d_model), jnp.float32)], axis=1)

    return pl.pallas_call(
        body,
        grid=(grid,),
        in_specs=[pl.BlockSpec((d_model, cols), lambda i: (0, i))],
        out_specs=pl.BlockSpec((cols, d_pad), lambda i: (i, 0)),
        out_shape=jax.ShapeDtypeStruct((vocab, d_pad), jnp.float32),
    )


def _gather_kernel(n_total, d_model, d_pad, chunk):
    b_per_w = n_total // _NW
    n_chunks = b_per_w // chunk
    mesh = plsc.VectorSubcoreMesh(core_axis_name="c", subcore_axis_name="s")

    @functools.partial(
        pl.kernel,
        mesh=mesh,
        out_type=jax.ShapeDtypeStruct((n_total, d_pad), jnp.float32),
        scratch_types=[
            pltpu.VMEM((b_per_w,), jnp.int32),
            pltpu.VMEM((_NB, chunk, d_model), jnp.float32),
            pltpu.SemaphoreType.DMA,
            pltpu.SemaphoreType.DMA,
        ],
        compiler_params=pltpu.CompilerParams(use_tc_tiling_on_sc=False),
    )
    def k(idx_hbm, table_hbm, out_hbm, idx_v, rows_v, gsem, ssem):
        wid = lax.axis_index("s") * _NC + lax.axis_index("c")
        base = wid * b_per_w

        def gather_args(i, b):
            return (table_hbm.at[idx_v.at[pl.ds(i * chunk, chunk)]],
                    rows_v.at[b], gsem)

        def store_args(i, b):
            return (rows_v.at[b],
                    out_hbm.at[pl.ds(base + i * chunk, chunk),
                               pl.ds(0, d_model)], ssem)

        def gather(i, b):
            pltpu.async_copy(*gather_args(i, b))

        def gather_wait(i, b):
            pltpu.make_async_copy(*gather_args(i, b)).wait()

        def store(i, b):
            pltpu.async_copy(*store_args(i, b))

        def store_wait(i, b):
            pltpu.make_async_copy(*store_args(i, b)).wait()

        pltpu.sync_copy(idx_hbm.at[pl.ds(base, b_per_w)], idx_v)
        for b in range(_NB):
            gather(b, b)

        @pl.loop(0, n_chunks - _NB, step=_NB)
        def _(i0):
            for b in range(_NB):
                i = i0 + b
                gather_wait(i, b)
                store(i, b)
                store_wait(i, b)
                gather(i + _NB, b)

        for b in range(_NB):
            i = n_chunks - _NB + b
            gather_wait(i, b)
            store(i, b)
        for b in range(_NB):
            store_wait(n_chunks - _NB + b, b)

    return k


def kernel(ids, emb_weight):
    batch, hist = ids.shape
    vocab, d_model = emb_weight.shape
    d_pad = 128
    n_total = batch * hist
    rows_per_pad = d_pad // d_model
    idx = ids.reshape(n_total).astype(jnp.int32) * rows_per_pad
    table_p = _pack_kernel(d_model, vocab, d_pad, 8192)(emb_weight.T)
    table_v = table_p.reshape(vocab * rows_per_pad, d_model)
    out = _gather_kernel(n_total, d_model, d_pad, 800)(idx, table_v)
    return out[:, :d_model].reshape(batch, hist, d_model)


# speedup vs baseline: 3.1112x; 1.0333x over previous
import functools

import jax
import jax.numpy as jnp
from jax import lax
from jax.experimental import pallas as pl
from jax.experimental.pallas import tpu as pltpu
from jax.experimental.pallas import tpu_sc as plsc

_NC = 2
_NS = 16
_NW = _NC * _NS
_NB = 2


def _pack_kernel(d_model, vocab, d_pad, cols):
    grid = (vocab + cols - 1) // cols

    def body(wt_ref, out_ref):
        t = jnp.transpose(wt_ref[...], (1, 0))
        out_ref[...] = jnp.concatenate(
            [t, jnp.zeros((cols, d_pad - d_model), jnp.float32)], axis=1)

    return pl.pallas_call(
        body,
        grid=(grid,),
        in_specs=[pl.BlockSpec((d_model, cols), lambda i: (0, i))],
        out_specs=pl.BlockSpec((cols, d_pad), lambda i: (i, 0)),
        out_shape=jax.ShapeDtypeStruct((vocab, d_pad), jnp.float32),
    )


def _gather_kernel(n_total, d_model, d_pad, chunk):
    b_per_w = n_total // _NW
    n_chunks = b_per_w // chunk
    mesh = plsc.VectorSubcoreMesh(core_axis_name="c", subcore_axis_name="s")

    @functools.partial(
        pl.kernel,
        mesh=mesh,
        out_type=jax.ShapeDtypeStruct((n_total, d_pad), jnp.float32),
        scratch_types=[
            pltpu.VMEM((b_per_w,), jnp.int32),
            pltpu.VMEM((_NB, chunk, d_model), jnp.float32),
            pltpu.SemaphoreType.DMA,
            pltpu.SemaphoreType.DMA,
        ],
        compiler_params=pltpu.CompilerParams(use_tc_tiling_on_sc=False),
    )
    def k(idx_hbm, table_hbm, out_hbm, idx_v, rows_v, gsem, ssem):
        wid = lax.axis_index("s") * _NC + lax.axis_index("c")
        base = wid * b_per_w

        def gather_args(i, b):
            return (table_hbm.at[idx_v.at[pl.ds(i * chunk, chunk)]],
                    rows_v.at[b], gsem)

        def store_args(i, b):
            return (rows_v.at[b],
                    out_hbm.at[pl.ds(base + i * chunk, chunk),
                               pl.ds(0, d_model)], ssem)

        def gather(i, b):
            pltpu.async_copy(*gather_args(i, b))

        def gather_wait(i, b):
            pltpu.make_async_copy(*gather_args(i, b)).wait()

        def store(i, b):
            pltpu.async_copy(*store_args(i, b))

        def store_wait(i, b):
            pltpu.make_async_copy(*store_args(i, b)).wait()

        pltpu.sync_copy(idx_hbm.at[pl.ds(base, b_per_w)], idx_v)
        for b in range(_NB):
            gather(b, b)

        @pl.loop(0, n_chunks - _NB, step=_NB)
        def _(i0):
            for b in range(_NB):
                i = i0 + b
                gather_wait(i, b)
                store(i, b)
                store_wait(i, b)
                gather(i + _NB, b)

        for b in range(_NB):
            i = n_chunks - _NB + b
            gather_wait(i, b)
            store(i, b)
        for b in range(_NB):
            store_wait(n_chunks - _NB + b, b)

    return k


def kernel(ids, emb_weight):
    batch, hist = ids.shape
    vocab, d_model = emb_weight.shape
    d_pad = 128
    n_total = batch * hist
    rows_per_pad = d_pad // d_model
    idx = ids.reshape(n_total).astype(jnp.int32) * rows_per_pad
    table_p = _pack_kernel(d_model, vocab, d_pad, 16384)(emb_weight.T)
    table_v = table_p.reshape(vocab * rows_per_pad, d_model)
    out = _gather_kernel(n_total, d_model, d_pad, 800)(idx, table_v)
    return out[:, :d_model].reshape(batch, hist, d_model)


# speedup vs baseline: 3.1444x; 1.0107x over previous
import functools

import jax
import jax.numpy as jnp
from jax import lax
from jax.experimental import pallas as pl
from jax.experimental.pallas import tpu as pltpu
from jax.experimental.pallas import tpu_sc as plsc

_NC = 2
_NS = 16
_NW = _NC * _NS
_NB = 2


def _pack_kernel(d_model, vocab, d_pad, cols):
    grid = (vocab + cols - 1) // cols

    def body(wt_ref, out_ref):
        t = jnp.transpose(wt_ref[...], (1, 0))
        out_ref[...] = jnp.concatenate(
            [t, jnp.zeros((cols, d_pad - d_model), jnp.float32)], axis=1)

    return pl.pallas_call(
        body,
        grid=(grid,),
        in_specs=[pl.BlockSpec((d_model, cols), lambda i: (0, i))],
        out_specs=pl.BlockSpec((cols, d_pad), lambda i: (i, 0)),
        out_shape=jax.ShapeDtypeStruct((vocab, d_pad), jnp.float32),
    )


def _gather_kernel(n_total, d_model, d_pad, chunk):
    b_per_w = n_total // _NW
    n_chunks = b_per_w // chunk
    mesh = plsc.VectorSubcoreMesh(core_axis_name="c", subcore_axis_name="s")

    @functools.partial(
        pl.kernel,
        mesh=mesh,
        out_type=jax.ShapeDtypeStruct((n_total, d_pad), jnp.float32),
        scratch_types=[
            pltpu.VMEM((b_per_w,), jnp.int32),
            pltpu.VMEM((_NB, chunk, d_model), jnp.float32),
            pltpu.SemaphoreType.DMA,
            pltpu.SemaphoreType.DMA,
        ],
        compiler_params=pltpu.CompilerParams(use_tc_tiling_on_sc=False),
    )
    def k(idx_hbm, table_hbm, out_hbm, idx_v, rows_v, gsem, ssem):
        wid = lax.axis_index("s") * _NC + lax.axis_index("c")
        base = wid * b_per_w

        def gather_args(i, b):
            return (table_hbm.at[idx_v.at[pl.ds(i * chunk, chunk)]],
                    rows_v.at[b], gsem)

        def store_args(i, b):
            return (rows_v.at[b],
                    out_hbm.at[pl.ds(base + i * chunk, chunk),
                               pl.ds(0, d_model)], ssem)

        def gather(i, b):
            pltpu.async_copy(*gather_args(i, b))

        def gather_wait(i, b):
            pltpu.make_async_copy(*gather_args(i, b)).wait()

        def store(i, b):
            pltpu.async_copy(*store_args(i, b))

        def store_wait(i, b):
            pltpu.make_async_copy(*store_args(i, b)).wait()

        pltpu.sync_copy(idx_hbm.at[pl.ds(base, b_per_w)], idx_v)
        for b in range(_NB):
            gather(b, b)

        @pl.loop(0, n_chunks - _NB, step=_NB)
        def _(i0):
            for b in range(_NB):
                i = i0 + b
                gather_wait(i, b)
                store(i, b)
                store_wait(i, b)
                gather(i + _NB, b)

        for b in range(_NB):
            i = n_chunks - _NB + b
            gather_wait(i, b)
            store(i, b)
        for b in range(_NB):
            store_wait(n_chunks - _NB + b, b)

    return k


def kernel(ids, emb_weight):
    batch, hist = ids.shape
    vocab, d_model = emb_weight.shape
    d_pad = 128
    n_total = batch * hist
    rows_per_pad = d_pad // d_model
    idx = ids.reshape(n_total).astype(jnp.int32) * rows_per_pad
    table_p = _pack_kernel(d_model, vocab, d_pad, 32768)(emb_weight.T)
    table_v = table_p.reshape(vocab * rows_per_pad, d_model)
    out = _gather_kernel(n_total, d_model, d_pad, 800)(idx, table_v)
    return out[:, :d_model].reshape(batch, hist, d_model)
